# baseline (speedup 1.0000x reference)
import functools

import jax
import jax.numpy as jnp
from jax import lax
from jax.experimental import pallas as pl
from jax.experimental.pallas import tpu as pltpu
from jax.experimental.pallas import tpu_sc as plsc

_LANES = 16
_CHUNK = 4096
_TILE = 128
_UNROLL = 8


@functools.lru_cache(maxsize=None)
def _build(n_rows: int, n_tab: int):
    info = plsc.get_sparse_core_info()
    nc, ns = info.num_cores, info.num_subcores
    nw = nc * ns
    n_bulk = n_rows // _CHUNK
    tail_base = n_bulk * _CHUNK
    tail_pad = (-(n_rows - tail_base)) % _TILE
    tail_rows = n_rows - tail_base + tail_pad
    tail_valid = n_rows - tail_base
    max_chunks = (n_bulk + nw - 1) // nw

    mesh = plsc.VectorSubcoreMesh(core_axis_name="c", subcore_axis_name="s")

    @functools.partial(
        pl.kernel,
        mesh=mesh,
        out_type=jax.ShapeDtypeStruct((2, n_rows), jnp.float32),
        scratch_types=[
            pltpu.VMEM((2, n_tab), jnp.float32),
            pltpu.VMEM((_CHUNK,), jnp.int32),
            pltpu.VMEM((_CHUNK,), jnp.int32),
            pltpu.VMEM((2, _CHUNK), jnp.float32),
            pltpu.VMEM((2, _CHUNK), jnp.float32),
            pltpu.VMEM((2, _CHUNK), jnp.float32),
            pltpu.VMEM((2, _CHUNK), jnp.float32),
            pltpu.SemaphoreType.DMA,
            pltpu.SemaphoreType.DMA,
            pltpu.SemaphoreType.DMA,
            pltpu.SemaphoreType.DMA,
        ],
        compiler_params=pltpu.CompilerParams(needs_layout_passes=False),
    )
    def k(dp_hbm, nums_hbm, tab_hbm, out_hbm, tab_v,
          nums0, nums1, dpb0, dpb1, ob0, ob1, si0, si1, so0, so1):
        wid = lax.axis_index("s") * nc + lax.axis_index("c")
        pltpu.sync_copy(tab_hbm, tab_v)

        nums_b, dp_b, out_b = (nums0, nums1), (dpb0, dpb1), (ob0, ob1)
        si, so = (si0, si1), (so0, so1)

        my_chunks = (n_bulk - wid + nw - 1) // nw

        def base_of(i):
            return (wid + i * nw) * _CHUNK

        def in_descs(i, b):
            base = base_of(i)
            return (
                pltpu.make_async_copy(
                    nums_hbm.at[pl.ds(base, _CHUNK)], nums_b[b], si[b]),
                pltpu.make_async_copy(
                    dp_hbm.at[:, pl.ds(base, _CHUNK)], dp_b[b], si[b]),
            )

        def out_desc(i, b):
            return pltpu.make_async_copy(
                out_b[b], out_hbm.at[:, pl.ds(base_of(i), _CHUNK)], so[b])

        def run_block(nums_v, dp_v, out_v, rows, clamp):
            @plsc.parallel_loop(0, rows, step=_LANES, unroll=_UNROLL)
            def vec_body(s):
                nd = nums_v[pl.ds(s, _LANES)]
                if clamp:
                    nd = jnp.minimum(jnp.maximum(nd, 0), n_tab - 1)
                t0 = plsc.load_gather(tab_v.at[0], [nd])
                t1 = plsc.load_gather(tab_v.at[1], [nd])
                m0 = jnp.exp(jnp.minimum(jnp.maximum(dp_v[0, pl.ds(s, _LANES)], -4.0), 4.0))
                m1 = jnp.exp(jnp.minimum(jnp.maximum(dp_v[1, pl.ds(s, _LANES)], -4.0), 4.0))
                out_v[0, pl.ds(s, _LANES)] = t0 * m0
                out_v[1, pl.ds(s, _LANES)] = t1 * m1

        for d in in_descs(0, 0):
            d.start()
        for i in range(max_chunks):
            b = i % 2

            if i + 1 < max_chunks:
                @pl.when(i + 1 < my_chunks)
                def _(i=i, b=b):
                    for d in in_descs(i + 1, 1 - b):
                        d.start()

            @pl.when(i < my_chunks)
            def _(i=i, b=b):
                for d in in_descs(i, b):
                    d.wait()
                if i >= 2:
                    out_desc(i - 2, b).wait()
                run_block(nums_b[b], dp_b[b], out_b[b], _CHUNK, clamp=False)
                out_desc(i, b).start()

        for i in range(max_chunks):
            @pl.when((i >= my_chunks - 2) & (i < my_chunks))
            def _(i=i):
                out_desc(i, i % 2).wait()

        if tail_rows:
            @pl.when(wid == nw - 1)
            def _():
                dyn_base = (wid - wid) + tail_base
                pltpu.sync_copy(nums_hbm.at[pl.ds(tail_base, tail_valid)],
                                nums0.at[pl.ds(0, tail_valid)])
                pltpu.sync_copy(dp_hbm.at[:, pl.ds(dyn_base, tail_rows)],
                                dpb0.at[:, pl.ds(0, tail_rows)])
                run_block(nums0, dpb0, ob0, tail_rows, clamp=True)
                pltpu.sync_copy(ob0.at[:, pl.ds(0, tail_rows)],
                                out_hbm.at[:, pl.ds(dyn_base, tail_rows)])

    return k


def kernel(disp_param, numbers, disp_param0):
    n_rows = disp_param.shape[0]
    n_tab = disp_param0.shape[0]
    nums = numbers.astype(jnp.int32)
    out_t = _build(n_rows, n_tab)(disp_param.T, nums, disp_param0.T)
    return out_t.T


# speedup vs baseline: 1.0807x; 1.0807x over previous
import functools

import jax
import jax.numpy as jnp
from jax import lax
from jax.experimental import pallas as pl
from jax.experimental.pallas import tpu as pltpu
from jax.experimental.pallas import tpu_sc as plsc

_LANES = 16
_CHUNK = 8192
_TILE = 128
_UNROLL = 8


@functools.lru_cache(maxsize=None)
def _build(n_rows: int, n_tab: int):
    info = plsc.get_sparse_core_info()
    nc, ns = info.num_cores, info.num_subcores
    nw = nc * ns
    n_bulk = n_rows // _CHUNK
    tail_base = n_bulk * _CHUNK
    tail_pad = (-(n_rows - tail_base)) % _TILE
    tail_rows = n_rows - tail_base + tail_pad
    tail_valid = n_rows - tail_base
    max_chunks = (n_bulk + nw - 1) // nw

    mesh = plsc.VectorSubcoreMesh(core_axis_name="c", subcore_axis_name="s")

    @functools.partial(
        pl.kernel,
        mesh=mesh,
        out_type=jax.ShapeDtypeStruct((2, n_rows), jnp.float32),
        scratch_types=[
            pltpu.VMEM((2, n_tab), jnp.float32),
            pltpu.VMEM((_CHUNK,), jnp.int32),
            pltpu.VMEM((_CHUNK,), jnp.int32),
            pltpu.VMEM((2, _CHUNK), jnp.float32),
            pltpu.VMEM((2, _CHUNK), jnp.float32),
            pltpu.VMEM((2, _CHUNK), jnp.float32),
            pltpu.VMEM((2, _CHUNK), jnp.float32),
            pltpu.SemaphoreType.DMA,
            pltpu.SemaphoreType.DMA,
            pltpu.SemaphoreType.DMA,
            pltpu.SemaphoreType.DMA,
        ],
        compiler_params=pltpu.CompilerParams(needs_layout_passes=False),
    )
    def k(dp_hbm, nums_hbm, tab_hbm, out_hbm, tab_v,
          nums0, nums1, dpb0, dpb1, ob0, ob1, si0, si1, so0, so1):
        wid = lax.axis_index("s") * nc + lax.axis_index("c")

        nums_b, dp_b, out_b = (nums0, nums1), (dpb0, dpb1), (ob0, ob1)
        si, so = (si0, si1), (so0, so1)

        my_chunks = (n_bulk - wid + nw - 1) // nw

        def base_of(i):
            return (wid + i * nw) * _CHUNK

        def in_descs(i, b):
            base = base_of(i)
            return (
                pltpu.make_async_copy(
                    nums_hbm.at[pl.ds(base, _CHUNK)], nums_b[b], si[b]),
                pltpu.make_async_copy(
                    dp_hbm.at[:, pl.ds(base, _CHUNK)], dp_b[b], si[b]),
            )

        def out_desc(i, b):
            return pltpu.make_async_copy(
                out_b[b], out_hbm.at[:, pl.ds(base_of(i), _CHUNK)], so[b])

        def run_block(nums_v, dp_v, out_v, rows, clamp):
            @plsc.parallel_loop(0, rows, step=_LANES, unroll=_UNROLL)
            def vec_body(s):
                nd = nums_v[pl.ds(s, _LANES)]
                if clamp:
                    nd = jnp.minimum(jnp.maximum(nd, 0), n_tab - 1)
                t0 = plsc.load_gather(tab_v.at[0], [nd])
                t1 = plsc.load_gather(tab_v.at[1], [nd])
                m0 = jnp.exp(jnp.minimum(jnp.maximum(dp_v[0, pl.ds(s, _LANES)], -4.0), 4.0))
                m1 = jnp.exp(jnp.minimum(jnp.maximum(dp_v[1, pl.ds(s, _LANES)], -4.0), 4.0))
                out_v[0, pl.ds(s, _LANES)] = t0 * m0
                out_v[1, pl.ds(s, _LANES)] = t1 * m1

        tab_copy = pltpu.make_async_copy(tab_hbm, tab_v, so0)
        tab_copy.start()
        for d in in_descs(0, 0):
            d.start()
        tab_copy.wait()
        for i in range(max_chunks):
            b = i % 2

            if i + 1 < max_chunks:
                @pl.when(i + 1 < my_chunks)
                def _(i=i, b=b):
                    for d in in_descs(i + 1, 1 - b):
                        d.start()

            @pl.when(i < my_chunks)
            def _(i=i, b=b):
                for d in in_descs(i, b):
                    d.wait()
                if i >= 2:
                    out_desc(i - 2, b).wait()
                run_block(nums_b[b], dp_b[b], out_b[b], _CHUNK, clamp=False)
                out_desc(i, b).start()

        for i in range(max_chunks):
            @pl.when((i >= my_chunks - 2) & (i < my_chunks))
            def _(i=i):
                out_desc(i, i % 2).wait()

        if tail_rows:
            @pl.when(wid == nw - 1)
            def _():
                dyn_base = (wid - wid) + tail_base
                pltpu.sync_copy(nums_hbm.at[pl.ds(tail_base, tail_valid)],
                                nums0.at[pl.ds(0, tail_valid)])
                pltpu.sync_copy(dp_hbm.at[:, pl.ds(dyn_base, tail_rows)],
                                dpb0.at[:, pl.ds(0, tail_rows)])
                run_block(nums0, dpb0, ob0, tail_rows, clamp=True)
                pltpu.sync_copy(ob0.at[:, pl.ds(0, tail_rows)],
                                out_hbm.at[:, pl.ds(dyn_base, tail_rows)])

    return k


def kernel(disp_param, numbers, disp_param0):
    n_rows = disp_param.shape[0]
    n_tab = disp_param0.shape[0]
    nums = numbers.astype(jnp.int32)
    out_t = _build(n_rows, n_tab)(disp_param.T, nums, disp_param0.T)
    return out_t.T
